# baseline (speedup 1.0000x reference)
import functools

import numpy as np
import jax
import jax.numpy as jnp
from jax import lax
from jax.experimental import pallas as pl
from jax.experimental.pallas import tpu as pltpu
from jax.experimental.pallas import tpu_sc as plsc

GRID = 64
NCELL = GRID * GRID * GRID
T_OUT = 512
NPTS = 10000

NCORES = 2
NTILES = 16
PTS_PER_TILE = 640
PTS_PAD = NTILES * PTS_PER_TILE
CHUNK = 128
NCHUNK = PTS_PER_TILE // CHUNK
GROUPS = PTS_PER_TILE // 16
CELLS_PER_TILE = NCELL // (NCORES * NTILES)
UNROLL = 8


def _interp_weights(L, size):
    pos = np.arange(size, dtype=np.float32) * np.float32((L - 1) / (size - 1))
    lo = np.clip(np.floor(pos).astype(np.int32), 0, L - 1)
    hi = np.minimum(lo + 1, L - 1)
    w = (pos - lo.astype(np.float32)).astype(np.float32)
    W = np.zeros((size, L), np.float32)
    W[np.arange(size), lo] += (1.0 - w)
    W[np.arange(size), hi] += w
    return W


_WV = _interp_weights(50, T_OUT)
_WP = _interp_weights(200, T_OUT)


def _sc_count(xs, ys, zs, zeros_hbm):
    mesh = plsc.VectorSubcoreMesh(core_axis_name="c", subcore_axis_name="s")

    @functools.partial(
        pl.kernel,
        mesh=mesh,
        out_type=jax.ShapeDtypeStruct((NCORES * NTILES, 16), jnp.float32),
        scratch_types=[
            pltpu.VMEM((PTS_PER_TILE,), jnp.float32),
            pltpu.VMEM((PTS_PER_TILE,), jnp.float32),
            pltpu.VMEM((PTS_PER_TILE,), jnp.float32),
            pltpu.VMEM((NCHUNK, CHUNK), jnp.int32),
            pltpu.VMEM((NCHUNK, CHUNK), jnp.float32),
            pltpu.VMEM((CELLS_PER_TILE,), jnp.float32),
            pltpu.VMEM((16,), jnp.float32),
            pltpu.VMEM_SHARED((NCELL,), jnp.float32),
        ],
    )
    def k(xs_hbm, ys_hbm, zs_hbm, zhbm, out_hbm, x_v, y_v, z_v,
          idx_v, val_v, red_v, acc_v, grid_sh):
        cid = lax.axis_index("c")
        sid = lax.axis_index("s")
        zero16 = jnp.zeros((16,), jnp.float32)
        cell0 = (cid * NTILES + sid) * CELLS_PER_TILE

        pltpu.sync_copy(zhbm, grid_sh.at[pl.ds(cell0, CELLS_PER_TILE)])
        plsc.subcore_barrier()

        sl = pl.ds(sid * PTS_PER_TILE, PTS_PER_TILE)
        pltpu.sync_copy(xs_hbm.at[sl], x_v)
        pltpu.sync_copy(ys_hbm.at[sl], y_v)
        pltpu.sync_copy(zs_hbm.at[sl], z_v)
        lanes = lax.iota(jnp.int32, 16)
        for g in range(GROUPS):
            x = x_v[pl.ds(g * 16, 16)]
            y = y_v[pl.ds(g * 16, 16)]
            z = z_v[pl.ds(g * 16, 16)]
            qx = jnp.clip(((x + 2.0) * 16.0).astype(jnp.int32), 0, GRID - 1)
            qy = jnp.clip(((y + 2.0) * 16.0).astype(jnp.int32), 0, GRID - 1)
            qz = jnp.clip(((z + 2.0) * 16.0).astype(jnp.int32), 0, GRID - 1)
            flat = qx * (GRID * GRID) + qy * GRID + qz
            gid = lanes + (g * 16) + sid * PTS_PER_TILE
            val = jnp.where(gid < NPTS, jnp.float32(1.0), jnp.float32(0.0))
            ch = g // (CHUNK // 16)
            off = (g % (CHUNK // 16)) * 16
            idx_v[ch, pl.ds(off, 16)] = flat
            val_v[ch, pl.ds(off, 16)] = val
        for chn in range(NCHUNK):
            pltpu.sync_copy(val_v.at[chn], grid_sh.at[idx_v.at[chn]],
                            add=True)

        plsc.subcore_barrier()

        pltpu.sync_copy(grid_sh.at[pl.ds(cell0, CELLS_PER_TILE)], red_v)

        def rbody(i, accs):
            base = i * (16 * UNROLL)
            out = []
            for j in range(UNROLL):
                v = red_v[pl.ds(base + j * 16, 16)]
                out.append(accs[j] + jnp.where(v > 0.0, jnp.float32(1.0),
                                               jnp.float32(0.0)))
            return tuple(out)

        accs = lax.fori_loop(0, CELLS_PER_TILE // (16 * UNROLL), rbody,
                             (zero16,) * UNROLL)
        acc = accs[0]
        for j in range(1, UNROLL):
            acc = acc + accs[j]
        acc_v[...] = acc
        pltpu.sync_copy(acc_v, out_hbm.at[cid * NTILES + sid])

    return k(xs, ys, zs, zeros_hbm)


def _tc_fuse(Wv, Wp, vision, proprio, imu):
    B = vision.shape[0]
    Lv = vision.shape[1]
    Lp = proprio.shape[1]
    Cv = vision.shape[2]
    Cp = proprio.shape[2]
    Ci = imu.shape[2]
    C_OUT = 512

    BB = 4

    def body(wv_ref, wp_ref, v_ref, p_ref, i_ref, o_ref):
        col = jnp.zeros((T_OUT, C_OUT - Cv - Cp - Ci), jnp.float32)
        for j in range(BB):
            va = jnp.dot(wv_ref[...], v_ref[j],
                         preferred_element_type=jnp.float32)
            pa = jnp.dot(wp_ref[...], p_ref[j],
                         preferred_element_type=jnp.float32)
            ia = jnp.dot(wp_ref[...], i_ref[j],
                         preferred_element_type=jnp.float32)
            o_ref[j] = jnp.concatenate([va, pa, ia, col], axis=-1)

    return pl.pallas_call(
        body,
        grid=(B // BB,),
        in_specs=[
            pl.BlockSpec((T_OUT, Lv), lambda b: (0, 0)),
            pl.BlockSpec((T_OUT, Lp), lambda b: (0, 0)),
            pl.BlockSpec((BB, Lv, Cv), lambda b: (b, 0, 0)),
            pl.BlockSpec((BB, Lp, Cp), lambda b: (b, 0, 0)),
            pl.BlockSpec((BB, Lp, Ci), lambda b: (b, 0, 0)),
        ],
        out_specs=pl.BlockSpec((BB, T_OUT, C_OUT), lambda b: (b, 0, 0)),
        out_shape=jax.ShapeDtypeStruct((B, T_OUT, C_OUT), jnp.float32),
        compiler_params=pltpu.CompilerParams(
            dimension_semantics=("arbitrary",)),
    )(Wv, Wp, vision, proprio, imu)


def _tc_patch_col(partial, out512, col_idx):
    B, T, C = out512.shape
    TILE0 = (col_idx // 128) * 128
    LANE = col_idx - TILE0

    def body(part_ref, in_ref, o_ref):
        s = jnp.sum(part_ref[...]) * np.float32(1.0 / NCELL)
        lane = lax.broadcasted_iota(jnp.int32, (B, T_OUT, 128), 2)
        o_ref[...] = jnp.where(lane == LANE, s, in_ref[...])

    return pl.pallas_call(
        body,
        grid=(1,),
        in_specs=[
            pl.BlockSpec((NCORES * NTILES, 16), lambda i: (0, 0)),
            pl.BlockSpec((B, T_OUT, 128), lambda i: (0, 0, TILE0 // 128)),
        ],
        out_specs=pl.BlockSpec((B, T_OUT, 128), lambda i: (0, 0, TILE0 // 128)),
        out_shape=jax.ShapeDtypeStruct((B, T, C), jnp.float32),
        input_output_aliases={1: 0},
    )(partial, out512)


def kernel(vision, proprio, imu, target_times, points):
    pts = jnp.pad(points, ((0, PTS_PAD - points.shape[0]), (0, 0)))
    zeros_hbm = jnp.zeros((CELLS_PER_TILE,), jnp.float32)
    partial = _sc_count(pts[:, 0], pts[:, 1], pts[:, 2], zeros_hbm)
    out512 = _tc_fuse(jnp.asarray(_WV), jnp.asarray(_WP),
                      vision, proprio, imu)
    C = vision.shape[2] + proprio.shape[2] + imu.shape[2] + 1
    out512 = _tc_patch_col(partial, out512, C - 1)
    return out512[:, :, :C]


# speedup vs baseline: 1.0496x; 1.0496x over previous
import functools

import numpy as np
import jax
import jax.numpy as jnp
from jax import lax
from jax.experimental import pallas as pl
from jax.experimental.pallas import tpu as pltpu
from jax.experimental.pallas import tpu_sc as plsc

GRID = 64
NCELL = GRID * GRID * GRID
T_OUT = 512
NPTS = 10000

NCORES = 2
NTILES = 16
PTS_PER_TILE = 640
PTS_PAD = NTILES * PTS_PER_TILE
CHUNK = 128
NCHUNK = PTS_PER_TILE // CHUNK
GROUPS = PTS_PER_TILE // 16
CELLS_PER_TILE = NCELL // (NCORES * NTILES)
UNROLL = 8


def _interp_weights(L, size):
    pos = np.arange(size, dtype=np.float32) * np.float32((L - 1) / (size - 1))
    lo = np.clip(np.floor(pos).astype(np.int32), 0, L - 1)
    hi = np.minimum(lo + 1, L - 1)
    w = (pos - lo.astype(np.float32)).astype(np.float32)
    W = np.zeros((size, L), np.float32)
    W[np.arange(size), lo] += (1.0 - w)
    W[np.arange(size), hi] += w
    return W


_WV = _interp_weights(50, T_OUT)
_WP = _interp_weights(200, T_OUT)


def _sc_count(xs, ys, zs, zeros_hbm):
    mesh = plsc.VectorSubcoreMesh(core_axis_name="c", subcore_axis_name="s")

    @functools.partial(
        pl.kernel,
        mesh=mesh,
        out_type=jax.ShapeDtypeStruct((NCORES * NTILES, 16), jnp.float32),
        scratch_types=[
            pltpu.VMEM((PTS_PER_TILE,), jnp.float32),
            pltpu.VMEM((PTS_PER_TILE,), jnp.float32),
            pltpu.VMEM((PTS_PER_TILE,), jnp.float32),
            pltpu.VMEM((NCHUNK, CHUNK), jnp.int32),
            pltpu.VMEM((NCHUNK, CHUNK), jnp.float32),
            pltpu.VMEM((CELLS_PER_TILE,), jnp.float32),
            pltpu.VMEM((16,), jnp.float32),
            pltpu.VMEM_SHARED((NCELL,), jnp.float32),
        ],
    )
    def k(xs_hbm, ys_hbm, zs_hbm, zhbm, out_hbm, x_v, y_v, z_v,
          idx_v, val_v, red_v, acc_v, grid_sh):
        cid = lax.axis_index("c")
        sid = lax.axis_index("s")
        zero16 = jnp.zeros((16,), jnp.float32)
        cell0 = (cid * NTILES + sid) * CELLS_PER_TILE

        pltpu.sync_copy(zhbm, grid_sh.at[pl.ds(cell0, CELLS_PER_TILE)])
        plsc.subcore_barrier()

        sl = pl.ds(sid * PTS_PER_TILE, PTS_PER_TILE)
        pltpu.sync_copy(xs_hbm.at[sl], x_v)
        pltpu.sync_copy(ys_hbm.at[sl], y_v)
        pltpu.sync_copy(zs_hbm.at[sl], z_v)
        lanes = lax.iota(jnp.int32, 16)
        for g in range(GROUPS):
            x = x_v[pl.ds(g * 16, 16)]
            y = y_v[pl.ds(g * 16, 16)]
            z = z_v[pl.ds(g * 16, 16)]
            qx = jnp.clip(((x + 2.0) * 16.0).astype(jnp.int32), 0, GRID - 1)
            qy = jnp.clip(((y + 2.0) * 16.0).astype(jnp.int32), 0, GRID - 1)
            qz = jnp.clip(((z + 2.0) * 16.0).astype(jnp.int32), 0, GRID - 1)
            flat = qx * (GRID * GRID) + qy * GRID + qz
            gid = lanes + (g * 16) + sid * PTS_PER_TILE
            val = jnp.where(gid < NPTS, jnp.float32(1.0), jnp.float32(0.0))
            ch = g // (CHUNK // 16)
            off = (g % (CHUNK // 16)) * 16
            idx_v[ch, pl.ds(off, 16)] = flat
            val_v[ch, pl.ds(off, 16)] = val
        for chn in range(NCHUNK):
            pltpu.sync_copy(val_v.at[chn], grid_sh.at[idx_v.at[chn]],
                            add=True)

        plsc.subcore_barrier()

        pltpu.sync_copy(grid_sh.at[pl.ds(cell0, CELLS_PER_TILE)], red_v)

        def rbody(i, accs):
            base = i * (16 * UNROLL)
            out = []
            for j in range(UNROLL):
                v = red_v[pl.ds(base + j * 16, 16)]
                out.append(accs[j] + jnp.where(v > 0.0, jnp.float32(1.0),
                                               jnp.float32(0.0)))
            return tuple(out)

        accs = lax.fori_loop(0, CELLS_PER_TILE // (16 * UNROLL), rbody,
                             (zero16,) * UNROLL)
        acc = accs[0]
        for j in range(1, UNROLL):
            acc = acc + accs[j]
        acc_v[...] = acc
        pltpu.sync_copy(acc_v, out_hbm.at[cid * NTILES + sid])

    return k(xs, ys, zs, zeros_hbm)


def _tc_fuse(Wv, Wp, partial, vision, proprio, imu):
    B = vision.shape[0]
    Lv = vision.shape[1]
    Lp = proprio.shape[1]
    Cv = vision.shape[2]
    Cp = proprio.shape[2]
    Ci = imu.shape[2]
    C_OUT = 512

    BB = 8

    def body(wv_ref, wp_ref, part_ref, v_ref, p_ref, i_ref, o_ref):
        s = jnp.sum(part_ref[...]) * np.float32(1.0 / NCELL)
        col = jnp.full((T_OUT, C_OUT - Cv - Cp - Ci), s, jnp.float32)
        for j in range(BB):
            va = jnp.dot(wv_ref[...], v_ref[j],
                         preferred_element_type=jnp.float32)
            pa = jnp.dot(wp_ref[...], p_ref[j],
                         preferred_element_type=jnp.float32)
            ia = jnp.dot(wp_ref[...], i_ref[j],
                         preferred_element_type=jnp.float32)
            o_ref[j] = jnp.concatenate([va, pa, ia, col], axis=-1)

    return pl.pallas_call(
        body,
        grid=(B // BB,),
        in_specs=[
            pl.BlockSpec((T_OUT, Lv), lambda b: (0, 0)),
            pl.BlockSpec((T_OUT, Lp), lambda b: (0, 0)),
            pl.BlockSpec((NCORES * NTILES, 16), lambda b: (0, 0)),
            pl.BlockSpec((BB, Lv, Cv), lambda b: (b, 0, 0)),
            pl.BlockSpec((BB, Lp, Cp), lambda b: (b, 0, 0)),
            pl.BlockSpec((BB, Lp, Ci), lambda b: (b, 0, 0)),
        ],
        out_specs=pl.BlockSpec((BB, T_OUT, C_OUT), lambda b: (b, 0, 0)),
        out_shape=jax.ShapeDtypeStruct((B, T_OUT, C_OUT), jnp.float32),
        compiler_params=pltpu.CompilerParams(
            dimension_semantics=("arbitrary",)),
    )(Wv, Wp, partial, vision, proprio, imu)


def kernel(vision, proprio, imu, target_times, points):
    pts = jnp.pad(points, ((0, PTS_PAD - points.shape[0]), (0, 0)))
    zeros_hbm = jnp.zeros((CELLS_PER_TILE,), jnp.float32)
    partial = _sc_count(pts[:, 0], pts[:, 1], pts[:, 2], zeros_hbm)
    out = _tc_fuse(jnp.asarray(_WV), jnp.asarray(_WP), partial,
                   vision, proprio, imu)
    C = vision.shape[2] + proprio.shape[2] + imu.shape[2] + 1
    return out[:, :, :C]
